# Initial kernel scaffold; baseline (speedup 1.0000x reference)
import jax
import jax.numpy as jnp
from jax.experimental import pallas as pl


def kernel(embeddings, pos_table):
    raise NotImplementedError("write your pallas kernel here")


# speedup vs baseline: 1.5043x; 1.5043x over previous
import jax
import jax.numpy as jnp
from jax.experimental import pallas as pl
from jax.experimental.pallas import tpu as pltpu

SEQ_BLOCK = 512


def _add_kernel(emb_ref, pos_ref, out_ref):
    out_ref[...] = emb_ref[...] + pos_ref[...]


def kernel(embeddings, pos_table):
    batch, seq, hid = embeddings.shape
    grid = (seq // SEQ_BLOCK, batch)
    return pl.pallas_call(
        _add_kernel,
        grid=grid,
        in_specs=[
            pl.BlockSpec((1, SEQ_BLOCK, hid), lambda i, j: (j, i, 0)),
            pl.BlockSpec((SEQ_BLOCK, hid), lambda i, j: (i, 0)),
        ],
        out_specs=pl.BlockSpec((1, SEQ_BLOCK, hid), lambda i, j: (j, i, 0)),
        out_shape=jax.ShapeDtypeStruct((batch, seq, hid), embeddings.dtype),
        compiler_params=pltpu.CompilerParams(
            dimension_semantics=("arbitrary", "arbitrary"),
        ),
    )(embeddings, pos_table)


# speedup vs baseline: 1.7394x; 1.1563x over previous
import jax
import jax.numpy as jnp
from jax.experimental import pallas as pl
from jax.experimental.pallas import tpu as pltpu

SEQ_BLOCK = 2048


def _add_kernel(emb_ref, pos_ref, out_ref):
    out_ref[...] = emb_ref[...] + pos_ref[...]


def kernel(embeddings, pos_table):
    batch, seq, hid = embeddings.shape
    grid = (seq // SEQ_BLOCK, batch)
    return pl.pallas_call(
        _add_kernel,
        grid=grid,
        in_specs=[
            pl.BlockSpec((1, SEQ_BLOCK, hid), lambda i, j: (j, i, 0)),
            pl.BlockSpec((SEQ_BLOCK, hid), lambda i, j: (i, 0)),
        ],
        out_specs=pl.BlockSpec((1, SEQ_BLOCK, hid), lambda i, j: (j, i, 0)),
        out_shape=jax.ShapeDtypeStruct((batch, seq, hid), embeddings.dtype),
        compiler_params=pltpu.CompilerParams(
            dimension_semantics=("arbitrary", "arbitrary"),
        ),
    )(embeddings, pos_table)
